# Initial kernel scaffold; baseline (speedup 1.0000x reference)
import jax
import jax.numpy as jnp
from jax.experimental import pallas as pl


def kernel(anchors, bbox_score, bbox_coeff):
    raise NotImplementedError("write your pallas kernel here")


# speedup vs baseline: 19.1906x; 19.1906x over previous
import functools

import jax
import jax.numpy as jnp
from jax import lax
from jax.experimental import pallas as pl
from jax.experimental.pallas import tpu as pltpu
from jax.experimental.pallas import tpu_sc as plsc

IMG = 800.0
PRE = 2000
POST = 300
THR = 0.7
NA = 20000
B = 8
SLOTS = 16
RS = 2048
NS = SLOTS * RS
LANES = B * SLOTS
BLK = 256
NBLK = RS // BLK
OUTW = 304

import numpy as np

_I32_MIN = np.int32(-2147483647 - 1)
_MASK31 = np.int32(0x7FFFFFFF)


def _ce(key, idx, s, axis, asc, pos):
    n = (RS, LANES)[axis]
    pkd = pltpu.roll(key, n - s, axis)
    pku = pltpu.roll(key, s, axis)
    pid2 = pltpu.roll(idx, n - s, axis)
    piu = pltpu.roll(idx, s, axis)
    is_lo = (pos & s) == 0
    pk = jnp.where(is_lo, pkd, pku)
    pi = jnp.where(is_lo, pid2, piu)
    me_wins = (key > pk) | ((key == pk) & (idx < pi))
    keep = ~(jnp.logical_xor(jnp.logical_xor(me_wins, is_lo), asc))
    return jnp.where(keep, key, pk), jnp.where(keep, idx, pi)


def _stage_list():
    st = []
    m = 2
    while m <= RS:
        s = m // 2
        while s >= 1:
            st.append(("p1", s, m))
            s //= 2
        m *= 2
    for r in range(4):
        st.append(("half", 1 << r, 0))
        s = RS // 2
        while s >= 1:
            st.append(("p2", s, r))
            s //= 2
    return st


def _apply_stage(key, idx, st, row, lane):
    kind, a, b = st
    if kind == "half":
        pk = pltpu.roll(key, LANES - a, 1)
        pi = pltpu.roll(idx, LANES - a, 1)
        win = (key > pk) | ((key == pk) & (idx < pi))
        return jnp.where(win, key, pk), jnp.where(win, idx, pi)
    if kind == "p1":
        asc = jnp.logical_xor(((row // b) & 1) == 1, (lane & 1) == 1)
    else:
        if b < 3:
            asc = ((lane >> (b + 1)) & 1) == 1
        else:
            asc = jnp.zeros((RS, LANES), jnp.bool_)
    return _ce(key, idx, a, 0, asc, row)


def _mk_part(group, first, last):
    def body(*refs):
        row = lax.broadcasted_iota(jnp.int32, (RS, LANES), 0)
        lane = lax.broadcasted_iota(jnp.int32, (RS, LANES), 1)
        if first:
            sc_ref = refs[0]
            elem = (lane & 15) * RS + row
            bits = pltpu.bitcast(sc_ref[...], jnp.int32)
            key = jnp.where(bits < 0, bits ^ _MASK31, bits)
            key = jnp.where(elem < NA, key, _I32_MIN)
            idx = elem
        else:
            key = refs[0][...]
            idx = refs[1][...]
        for st in group:
            key, idx = _apply_stage(key, idx, st, row, lane)
        if last:
            obits = jnp.where(key < 0, key ^ _MASK31, key)
            refs[-2][...] = pltpu.bitcast(obits, jnp.float32)
            refs[-1][...] = idx + (lane >> 4) * NA
        else:
            refs[-2][...] = key
            refs[-1][...] = idx
    return body


def _topk(scores_pad_t, part_size=20):
    stages = _stage_list()
    groups = [stages[i:i + part_size] for i in range(0, len(stages), part_size)]
    cur = (scores_pad_t,)
    for gi, group in enumerate(groups):
        first = gi == 0
        last = gi == len(groups) - 1
        odt = jnp.float32 if last else jnp.int32
        cur = pl.pallas_call(
            _mk_part(group, first, last),
            out_shape=[
                jax.ShapeDtypeStruct((RS, LANES), odt),
                jax.ShapeDtypeStruct((RS, LANES), jnp.int32),
            ],
        )(*cur)
    return cur


_NW = 32
_BPW = (B * RS) // _NW


def _sc_gather_body(tbl_hbm, idx_hbm, out_hbm, idx_v, rows_v, sem):
    wid = lax.axis_index("s") * 2 + lax.axis_index("c")
    base = wid * _BPW
    pltpu.sync_copy(idx_hbm.at[pl.ds(base, _BPW)], idx_v)
    pltpu.async_copy(tbl_hbm.at[idx_v], rows_v, sem).wait()
    pltpu.sync_copy(rows_v, out_hbm.at[pl.ds(base, _BPW)])


def _sc_gather(tbl, idx_flat):
    k = functools.partial(
        pl.kernel,
        out_type=jax.ShapeDtypeStruct((B * RS, 16), jnp.float32),
        mesh=plsc.VectorSubcoreMesh(core_axis_name="c", subcore_axis_name="s"),
        scratch_types=[
            pltpu.VMEM((_BPW,), jnp.int32),
            pltpu.VMEM((_BPW, 16), jnp.float32),
            pltpu.SemaphoreType.DMA,
        ],
        compiler_params=pltpu.CompilerParams(use_tc_tiling_on_sc=False),
    )(_sc_gather_body)
    return k(tbl, idx_flat)


def _transform_rows(ax1, ay1, ax2, ay2, dx, dy, dw, dh):
    w = ax2 - ax1 + 1.0
    h = ay2 - ay1 + 1.0
    cx = ax1 + 0.5 * w
    cy = ay1 + 0.5 * h
    pcx = dx * w + cx
    pcy = dy * h + cy
    pw = jnp.exp(dw) * w
    ph = jnp.exp(dh) * h
    x1 = jnp.clip(pcx - 0.5 * pw, 0.0, IMG - 1.0)
    y1 = jnp.clip(pcy - 0.5 * ph, 0.0, IMG - 1.0)
    x2 = jnp.clip(pcx + 0.5 * pw, 0.0, IMG - 1.0)
    y2 = jnp.clip(pcy + 0.5 * ph, 0.0, IMG - 1.0)
    return x1, y1, x2, y2


def _cumsum_lanes(x):
    lane = lax.broadcasted_iota(jnp.int32, (1, RS), 1)
    s = 1
    while s < RS:
        sh = pltpu.roll(x, s, 1)
        x = x + jnp.where(lane >= s, sh, 0)
        s *= 2
    return x


def _nms_kernel(g_ref, gt_ref, sc_ref, out_ref):
    g = g_ref[0]
    gt = gt_ref[0]
    scores = sc_ref[0]

    x1r, y1r, x2r, y2r = _transform_rows(
        gt[0:1], gt[1:2], gt[2:3], gt[3:4], gt[4:5], gt[5:6], gt[6:7], gt[7:8])
    area_r = (x2r - x1r + 1.0) * (y2r - y1r + 1.0)
    x1c, y1c, x2c, y2c = _transform_rows(
        g[:, 0:1], g[:, 1:2], g[:, 2:3], g[:, 3:4],
        g[:, 4:5], g[:, 5:6], g[:, 6:7], g[:, 7:8])
    area_c = (x2c - x1c + 1.0) * (y2c - y1c + 1.0)

    cols = lax.broadcasted_iota(jnp.int32, (1, RS), 1)
    keep_row = cols < PRE
    ident = (lax.broadcasted_iota(jnp.int32, (BLK, BLK), 0)
             == lax.broadcasted_iota(jnp.int32, (BLK, BLK), 1)).astype(jnp.float32)
    lrow = lax.broadcasted_iota(jnp.int32, (BLK, BLK), 0)
    lcol = lax.broadcasted_iota(jnp.int32, (BLK, BLK), 1)
    up_tri = lcol > lrow
    lo_tri = lcol < lrow

    for i in range(NBLK):
        lo = i * BLK
        bsl = slice(lo, lo + BLK)
        bx1, by1, bx2, by2 = x1c[bsl], y1c[bsl], x2c[bsl], y2c[bsl]
        barea = area_c[bsl]
        xx1 = jnp.maximum(bx1, x1r)
        yy1 = jnp.maximum(by1, y1r)
        xx2 = jnp.minimum(bx2, x2r)
        yy2 = jnp.minimum(by2, y2r)
        inter = (jnp.maximum(xx2 - xx1 + 1.0, 0.0)
                 * jnp.maximum(yy2 - yy1 + 1.0, 0.0))
        iou = inter / (barea + area_r - inter)
        osup = iou > THR
        oblk = osup[:, bsl]
        o_fwd = oblk & up_tri
        o_bwd = oblk & lo_tri

        init_row = keep_row[:, bsl]
        init_col = lax.dot_general(
            ident, init_row.astype(jnp.float32),
            (((1,), (1,)), ((), ())),
            preferred_element_type=jnp.float32) > 0.5

        def body(st):
            kc, kr, _, it = st
            supc = jnp.any(o_bwd & (kr > 0), axis=1, keepdims=True)
            supr = jnp.any(o_fwd & (kc > 0), axis=0, keepdims=True)
            nkc = (init_col & ~supc).astype(jnp.int32)
            nkr = (init_row & ~supr).astype(jnp.int32)
            changed = jnp.any(nkc != kc).astype(jnp.int32)
            return nkc, nkr, changed, it + 1

        def cond(st):
            _, _, changed, it = st
            return (changed > 0) & (it < BLK + 2)

        kc_i, kr_i, _, _ = lax.while_loop(
            cond, body,
            (init_col.astype(jnp.int32), init_row.astype(jnp.int32),
             jnp.int32(1), jnp.int32(0)))
        kc = kc_i > 0
        kr = kr_i > 0

        rows_glob = lax.broadcasted_iota(jnp.int32, (BLK, 1), 0) + lo
        later = cols > rows_glob
        sup_later = jnp.any(osup & later & kc, axis=0, keepdims=True)
        pieces = []
        if lo > 0:
            pieces.append(keep_row[:, :lo])
        pieces.append(kr)
        if lo + BLK < RS:
            pieces.append(keep_row[:, lo + BLK:])
        keep_row = jnp.concatenate(pieces, axis=1)
        keep_row = keep_row & ~sup_later

    valid = cols < PRE
    keep = keep_row & valid
    keep_i = keep.astype(jnp.int32)
    sup_i = (valid & ~keep).astype(jnp.int32)
    pref_k = _cumsum_lanes(keep_i)
    pref_s = _cumsum_lanes(sup_i)
    nkeep = jnp.sum(keep_i)
    rank = jnp.where(keep, pref_k - keep_i, nkeep + pref_s - sup_i)
    rank = jnp.where(valid, rank, 2 * RS)

    p_iota = lax.broadcasted_iota(jnp.int32, (OUTW, 1), 0)
    onehot = (rank == p_iota).astype(jnp.float32)
    zeros3 = jnp.zeros((3, RS), jnp.float32)
    vals8 = jnp.concatenate(
        [scores, x1r, y1r, x2r, y2r, zeros3], axis=0)
    res = lax.dot_general(
        onehot, vals8, (((1,), (1,)), ((), ())),
        preferred_element_type=jnp.float32)
    out_ref[0] = res


def _nms(g3, gt3, scores3):
    return pl.pallas_call(
        _nms_kernel,
        grid=(B,),
        in_specs=[
            pl.BlockSpec((1, RS, 16), lambda i: (i, 0, 0)),
            pl.BlockSpec((1, 16, RS), lambda i: (i, 0, 0)),
            pl.BlockSpec((1, 1, RS), lambda i: (i, 0, 0)),
        ],
        out_specs=pl.BlockSpec((1, OUTW, 8), lambda i: (i, 0, 0)),
        out_shape=jax.ShapeDtypeStruct((B, OUTW, 8), jnp.float32),
    )(g3, gt3, scores3)


def kernel(anchors, bbox_score, bbox_coeff):
    fg = bbox_score[:, :, 0]
    pad = jnp.zeros((B, NS - NA), jnp.float32)
    sc_t = (jnp.concatenate([fg, pad], axis=1)
            .reshape(B, SLOTS, RS).transpose(2, 0, 1).reshape(RS, LANES))
    score_srt, idxg = _topk(sc_t)
    idx_flat = idxg[:, ::SLOTS].T.reshape(-1)
    scores3 = score_srt[:, ::SLOTS].T.reshape(B, 1, RS)

    anc_b = jnp.broadcast_to(anchors[None], (B, NA, 4))
    tbl = jnp.concatenate(
        [anc_b, bbox_coeff, jnp.zeros((B, NA, 8), jnp.float32)],
        axis=-1).reshape(B * NA, 16)
    g = _sc_gather(tbl, idx_flat)
    g3 = g.reshape(B, RS, 16)
    gt3 = g3.transpose(0, 2, 1)

    out = _nms(g3, gt3, scores3)
    return (out[:, :POST, 0:1], out[:, :POST, 1:5])
